# baseline (speedup 1.0000x reference)
import functools

import jax
import jax.numpy as jnp
from jax import lax
from jax.experimental import pallas as pl
from jax.experimental.pallas import tpu as pltpu
from jax.experimental.pallas import tpu_sc as plsc

B = 4096
DEG = 50
DEGP = 56
D = 64
NC = 2
NS = 16
NW = NC * NS
IDX_W = 128
ROWS_PER_W = (B * DEGP) // NW // IDX_W
EPW = ROWS_PER_W * IDX_W
NBUF = 3
GROUP = 4
GROWS = GROUP * IDX_W
NG = ROWS_PER_W // GROUP
UPW = B // NW

BB = 256
NBLK = BB * DEGP


def _sc_gather(item_emb, idx2, user_emb, nodes2):
    mesh = plsc.VectorSubcoreMesh(
        core_axis_name="c", subcore_axis_name="s",
        num_cores=NC, num_subcores=NS)

    @functools.partial(
        pl.kernel,
        out_type=(
            jax.ShapeDtypeStruct((B * DEGP, D), jnp.float32),
            jax.ShapeDtypeStruct((B, D), jnp.float32),
        ),
        mesh=mesh,
        compiler_params=pltpu.CompilerParams(use_tc_tiling_on_sc=False),
        scratch_types=(
            pltpu.VMEM((ROWS_PER_W, IDX_W), jnp.int32),
            pltpu.VMEM((NBUF, GROWS, D), jnp.float32),
            pltpu.VMEM((UPW,), jnp.int32),
            pltpu.VMEM((UPW, D), jnp.float32),
            pltpu.SemaphoreType.DMA((NBUF,)),
            pltpu.SemaphoreType.DMA((NBUF,)),
            pltpu.SemaphoreType.DMA,
        ),
    )
    def k(item_hbm, idx_hbm, user_hbm, nodes_hbm, g_hbm, u_hbm,
          idx_v, bufs, uidx_v, urows_v, gsems, osems, usem):
        wid = lax.axis_index("s") * NC + lax.axis_index("c")
        pltpu.sync_copy(idx_hbm.at[pl.ds(wid * ROWS_PER_W, ROWS_PER_W)], idx_v)
        pltpu.sync_copy(nodes_hbm.at[wid], uidx_v)
        ucp = pltpu.async_copy(user_hbm.at[uidx_v], urows_v, usem)
        obase = wid * EPW

        def fire_g(g, b):
            for j in range(GROUP):
                pltpu.async_copy(
                    item_hbm.at[idx_v.at[g * GROUP + j]],
                    bufs.at[b, pl.ds(j * IDX_W, IDX_W)], gsems.at[b])

        def drain_g(b):
            pltpu.make_async_copy(
                item_hbm.at[pl.ds(0, GROWS)], bufs.at[b], gsems.at[b]).wait()

        def fire_o(g, b):
            pltpu.async_copy(
                bufs.at[b], g_hbm.at[pl.ds(obase + g * GROWS, GROWS)],
                osems.at[b])

        def drain_o(b):
            pltpu.make_async_copy(
                bufs.at[b], g_hbm.at[pl.ds(obase, GROWS)], osems.at[b]).wait()

        fire_g(0, 0)
        fire_g(1, 1)
        drain_g(0)
        fire_o(0, 0)
        fire_g(2, 2)

        def body(g, carry):
            b = lax.rem(g, NBUF)
            bp = lax.rem(g + 2, NBUF)
            drain_g(b)
            fire_o(g, b)
            drain_o(bp)
            fire_g(g + 2, bp)
            return carry

        lax.fori_loop(1, NG - 2, body, 0)
        drain_g((NG - 2) % NBUF)
        fire_o(NG - 2, (NG - 2) % NBUF)
        drain_o((NG - 3) % NBUF)
        drain_g((NG - 1) % NBUF)
        fire_o(NG - 1, (NG - 1) % NBUF)
        drain_o((NG - 2) % NBUF)
        drain_o((NG - 1) % NBUF)
        ucp.wait()
        pltpu.sync_copy(urows_v, u_hbm.at[pl.ds(wid * UPW, UPW)])

    return k(item_emb, idx2, user_emb, nodes2)


def _tc_body(g_ref, rid_ref, u_ref, w1_ref, w1b_ref, w2_ref, w2b_ref,
             wa1_ref, wa1b_ref, wa2_ref, wa2b_ref, wa3_ref, wa3b_ref,
             remb_ref, out_ref):
    f32 = jnp.float32
    g = g_ref[...]
    ids = rid_ref[...]
    w1a = w1_ref[0:D, :]
    r1 = jnp.dot(remb_ref[...], w1_ref[D:2 * D, :],
                 preferred_element_type=f32)
    rc = jnp.zeros((NBLK, D), f32)
    for k in range(5):
        rc = rc + jnp.where(ids == k, f32(1.0), f32(0.0)) * r1[k:k + 1, :]

    t = jnp.maximum(jnp.dot(g, w1a, preferred_element_type=f32)
                    + rc + w1b_ref[...], 0.0)
    uv_r = jnp.maximum(jnp.dot(t, w2_ref[...], preferred_element_type=f32)
                       + w2b_ref[...], 0.0)

    self_c = jnp.dot(u_ref[...], wa1_ref[D:2 * D, :],
                     preferred_element_type=f32)
    h1 = jnp.dot(uv_r, wa1_ref[0:D, :], preferred_element_type=f32)
    h = jnp.maximum(h1.reshape(BB, DEGP, D) + self_c[:, None, :]
                    + wa1b_ref[...][None, :, :], 0.0)
    h2 = jnp.maximum(jnp.dot(h.reshape(NBLK, D), wa2_ref[...],
                             preferred_element_type=f32)
                     + wa2b_ref[...], 0.0)
    logits = (jnp.sum(h2 * wa3_ref[...], axis=-1, keepdims=True)
              + wa3b_ref[...])

    l3 = logits.reshape(BB, DEGP, 1)
    pos = lax.broadcasted_iota(jnp.int32, (BB, DEGP, 1), 1)
    valid = pos < DEG
    l3 = jnp.where(valid, l3, f32(-1e30))
    m = jnp.max(l3, axis=1, keepdims=True)
    e = jnp.exp(l3 - m)
    e = jnp.where(valid, e, f32(0.0))
    s = jnp.sum(e, axis=1, keepdims=True)
    att = e / s
    out_ref[...] = jnp.sum(uv_r.reshape(BB, DEGP, D) * att, axis=1)


def _tc_mlp(g, rid, u, w1_w, w1_b, w2_w, w2_b, wa1_w, wa1_b, wa2_w, wa2_b,
            wa3r, wa3_b, remb):
    grid = (B // BB,)
    full = lambda shape: pl.BlockSpec(shape, lambda i: (0, 0))
    return pl.pallas_call(
        _tc_body,
        grid=grid,
        in_specs=[
            pl.BlockSpec((NBLK, D), lambda i: (i, 0)),
            pl.BlockSpec((NBLK, 1), lambda i: (i, 0)),
            pl.BlockSpec((BB, D), lambda i: (i, 0)),
            full((2 * D, D)), full((1, D)),
            full((D, D)), full((1, D)),
            full((2 * D, D)), full((1, D)),
            full((D, D)), full((1, D)),
            full((1, D)), full((1, 1)),
            full((8, D)),
        ],
        out_specs=pl.BlockSpec((BB, D), lambda i: (i, 0)),
        out_shape=jax.ShapeDtypeStruct((B, D), jnp.float32),
    )(g, rid, u, w1_w, w1_b, w2_w, w2_b, wa1_w, wa1_b, wa2_w, wa2_b,
      wa3r, wa3_b, remb)


def kernel(nodes, uv_adjacency, ratings, user_emb, item_emb, rating_emb,
           w1_w, w1_b, w2_w, w2_b, wa1_w, wa1_b, wa2_w, wa2_b, wa3_w, wa3_b):
    adj_p = jnp.pad(uv_adjacency.astype(jnp.int32), ((0, 0), (0, DEGP - DEG)))
    idx2 = adj_p.reshape(B * DEGP // IDX_W, IDX_W)
    rat_p = jnp.pad(ratings.astype(jnp.int32), ((0, 0), (0, DEGP - DEG)))
    rid = rat_p.reshape(B * DEGP, 1)
    nodes2 = nodes.astype(jnp.int32).reshape(NW, UPW)

    g, u = _sc_gather(item_emb, idx2, user_emb, nodes2)

    remb = jnp.pad(rating_emb, ((0, 3), (0, 0)))
    return _tc_mlp(
        g, rid, u,
        w1_w, w1_b.reshape(1, D),
        w2_w, w2_b.reshape(1, D),
        wa1_w, wa1_b.reshape(1, D),
        wa2_w, wa2_b.reshape(1, D),
        wa3_w.reshape(1, D), wa3_b.reshape(1, 1),
        remb)


# speedup vs baseline: 2.0009x; 2.0009x over previous
import functools

import jax
import jax.numpy as jnp
from jax import lax
from jax.experimental import pallas as pl
from jax.experimental.pallas import tpu as pltpu
from jax.experimental.pallas import tpu_sc as plsc

B = 4096
DEG = 50
DEGP = 56
D = 64
NC = 2
NS = 16
NW = NC * NS
IDX_W = 128
ROWS_PER_W = (B * DEGP) // NW // IDX_W
EPW = ROWS_PER_W * IDX_W
NBUF = 3
GROUP = 4
GROWS = GROUP * IDX_W
NG = ROWS_PER_W // GROUP
UPW = B // NW

BB = 256
NBLK = BB * DEGP


def _sc_gather(item_emb, idx2, user_emb, nodes2):
    mesh = plsc.VectorSubcoreMesh(
        core_axis_name="c", subcore_axis_name="s",
        num_cores=NC, num_subcores=NS)

    @functools.partial(
        pl.kernel,
        out_type=(
            jax.ShapeDtypeStruct((B * DEGP, D), jnp.float32),
            jax.ShapeDtypeStruct((B, D), jnp.float32),
        ),
        mesh=mesh,
        compiler_params=pltpu.CompilerParams(use_tc_tiling_on_sc=False),
        scratch_types=(
            pltpu.VMEM((ROWS_PER_W, IDX_W), jnp.int32),
            pltpu.VMEM((NBUF, GROWS, D), jnp.float32),
            pltpu.VMEM((UPW,), jnp.int32),
            pltpu.VMEM((UPW, D), jnp.float32),
            pltpu.SemaphoreType.DMA((NBUF,)),
            pltpu.SemaphoreType.DMA((NBUF,)),
            pltpu.SemaphoreType.DMA,
        ),
    )
    def k(item_hbm, idx_hbm, user_hbm, nodes_hbm, g_hbm, u_hbm,
          idx_v, bufs, uidx_v, urows_v, gsems, osems, usem):
        wid = lax.axis_index("s") * NC + lax.axis_index("c")
        pltpu.sync_copy(idx_hbm.at[pl.ds(wid * ROWS_PER_W, ROWS_PER_W)], idx_v)
        pltpu.sync_copy(nodes_hbm.at[wid], uidx_v)
        ucp = pltpu.async_copy(user_hbm.at[uidx_v], urows_v, usem)
        obase = wid * EPW

        def fire_g(g, b):
            for j in range(GROUP):
                pltpu.async_copy(
                    item_hbm.at[idx_v.at[g * GROUP + j]],
                    bufs.at[b, pl.ds(j * IDX_W, IDX_W)], gsems.at[b])

        def drain_g(b):
            pltpu.make_async_copy(
                item_hbm.at[pl.ds(0, GROWS)], bufs.at[b], gsems.at[b]).wait()

        def fire_o(g, b):
            pltpu.async_copy(
                bufs.at[b], g_hbm.at[pl.ds(obase + g * GROWS, GROWS)],
                osems.at[b])

        def drain_o(b):
            pltpu.make_async_copy(
                bufs.at[b], g_hbm.at[pl.ds(obase, GROWS)], osems.at[b]).wait()

        fire_g(0, 0)
        fire_g(1, 1)
        drain_g(0)
        fire_o(0, 0)
        fire_g(2, 2)

        def body(g, carry):
            b = lax.rem(g, NBUF)
            bp = lax.rem(g + 2, NBUF)
            drain_g(b)
            fire_o(g, b)
            drain_o(bp)
            fire_g(g + 2, bp)
            return carry

        lax.fori_loop(1, NG - 2, body, 0)
        drain_g((NG - 2) % NBUF)
        fire_o(NG - 2, (NG - 2) % NBUF)
        drain_o((NG - 3) % NBUF)
        drain_g((NG - 1) % NBUF)
        fire_o(NG - 1, (NG - 1) % NBUF)
        drain_o((NG - 2) % NBUF)
        drain_o((NG - 1) % NBUF)
        ucp.wait()
        pltpu.sync_copy(urows_v, u_hbm.at[pl.ds(wid * UPW, UPW)])

    return k(item_emb, idx2, user_emb, nodes2)


def _tc_body(g_ref, rid_ref, u_ref, w1_ref, w1b_ref, w2_ref, w2b_ref,
             wa1_ref, wa1b_ref, wa2_ref, wa2b_ref, wa3_ref, wa3b_ref,
             remb_ref, out_ref):
    f32 = jnp.float32
    g = g_ref[...]
    ids = rid_ref[...]
    w1a = w1_ref[0:D, :]
    r1 = jnp.dot(remb_ref[...], w1_ref[D:2 * D, :],
                 preferred_element_type=f32)
    rc = jnp.zeros((NBLK, D), f32)
    for k in range(5):
        rc = rc + jnp.where(ids == k, f32(1.0), f32(0.0)) * r1[k:k + 1, :]

    t = jnp.maximum(jnp.dot(g, w1a, preferred_element_type=f32)
                    + rc + w1b_ref[...], 0.0)
    uv_r = jnp.maximum(jnp.dot(t, w2_ref[...], preferred_element_type=f32)
                       + w2b_ref[...], 0.0)

    self_c = jnp.dot(u_ref[...], wa1_ref[D:2 * D, :],
                     preferred_element_type=f32)
    h1 = jnp.dot(uv_r, wa1_ref[0:D, :], preferred_element_type=f32)
    h = jnp.maximum(h1.reshape(BB, DEGP, D) + self_c[:, None, :]
                    + wa1b_ref[...][None, :, :], 0.0)
    h2 = jnp.maximum(jnp.dot(h.reshape(NBLK, D), wa2_ref[...],
                             preferred_element_type=f32)
                     + wa2b_ref[...], 0.0)
    logits = (jnp.sum(h2 * wa3_ref[...], axis=-1, keepdims=True)
              + wa3b_ref[...])

    l3 = logits.reshape(BB, DEGP, 1)
    pos = lax.broadcasted_iota(jnp.int32, (BB, DEGP, 1), 1)
    valid = pos < DEG
    l3 = jnp.where(valid, l3, f32(-1e30))
    m = jnp.max(l3, axis=1, keepdims=True)
    e = jnp.exp(l3 - m)
    e = jnp.where(valid, e, f32(0.0))
    s = jnp.sum(e, axis=1, keepdims=True)
    att = e / s
    out_ref[...] = jnp.sum(uv_r.reshape(BB, DEGP, D) * att, axis=1)


def _tc_mlp(g, rid, u, w1_w, w1_b, w2_w, w2_b, wa1_w, wa1_b, wa2_w, wa2_b,
            wa3r, wa3_b, remb):
    grid = (B // BB,)
    full = lambda shape: pl.BlockSpec(shape, lambda i: (0, 0))
    return pl.pallas_call(
        _tc_body,
        grid=grid,
        in_specs=[
            pl.BlockSpec((NBLK, D), lambda i: (i, 0)),
            pl.BlockSpec((NBLK, 1), lambda i: (i, 0)),
            pl.BlockSpec((BB, D), lambda i: (i, 0)),
            full((2 * D, D)), full((1, D)),
            full((D, D)), full((1, D)),
            full((2 * D, D)), full((1, D)),
            full((D, D)), full((1, D)),
            full((1, D)), full((1, 1)),
            full((8, D)),
        ],
        out_specs=pl.BlockSpec((BB, D), lambda i: (i, 0)),
        out_shape=jax.ShapeDtypeStruct((B, D), jnp.float32),
    )(g, rid, u, w1_w, w1_b, w2_w, w2_b, wa1_w, wa1_b, wa2_w, wa2_b,
      wa3r, wa3_b, remb)


def kernel(nodes, uv_adjacency, ratings, user_emb, item_emb, rating_emb,
           w1_w, w1_b, w2_w, w2_b, wa1_w, wa1_b, wa2_w, wa2_b, wa3_w, wa3_b):
    npad = DEGP - DEG
    pad_idx = (lax.broadcasted_iota(jnp.int32, (B, npad), 0) * npad
               + lax.broadcasted_iota(jnp.int32, (B, npad), 1))
    adj_p = jnp.concatenate([uv_adjacency.astype(jnp.int32), pad_idx], axis=1)
    idx2 = adj_p.reshape(B * DEGP // IDX_W, IDX_W)
    rat_p = jnp.pad(ratings.astype(jnp.int32), ((0, 0), (0, DEGP - DEG)))
    rid = rat_p.reshape(B * DEGP, 1)
    nodes2 = nodes.astype(jnp.int32).reshape(NW, UPW)

    g, u = _sc_gather(item_emb, idx2, user_emb, nodes2)

    remb = jnp.pad(rating_emb, ((0, 3), (0, 0)))
    return _tc_mlp(
        g, rid, u,
        w1_w, w1_b.reshape(1, D),
        w2_w, w2_b.reshape(1, D),
        wa1_w, wa1_b.reshape(1, D),
        wa2_w, wa2_b.reshape(1, D),
        wa3_w.reshape(1, D), wa3_b.reshape(1, 1),
        remb)


# speedup vs baseline: 2.2758x; 1.1374x over previous
import functools

import jax
import jax.numpy as jnp
from jax import lax
from jax.experimental import pallas as pl
from jax.experimental.pallas import tpu as pltpu
from jax.experimental.pallas import tpu_sc as plsc

B = 4096
DEG = 50
DEGP = 56
D = 64
NC = 2
NS = 16
NW = NC * NS
IDX_W = 128
ROWS_PER_W = (B * DEGP) // NW // IDX_W
EPW = ROWS_PER_W * IDX_W
NBUF = 3
GROUP = 4
GROWS = GROUP * IDX_W
NG = ROWS_PER_W // GROUP
UPW = B // NW

BB = 256
NBLK = BB * DEGP


def _sc_gather(item_emb, idx2, user_emb, nodes2):
    mesh = plsc.VectorSubcoreMesh(
        core_axis_name="c", subcore_axis_name="s",
        num_cores=NC, num_subcores=NS)

    @functools.partial(
        pl.kernel,
        out_type=(
            jax.ShapeDtypeStruct((B * DEGP, D), jnp.float32),
            jax.ShapeDtypeStruct((B, D), jnp.float32),
        ),
        mesh=mesh,
        compiler_params=pltpu.CompilerParams(use_tc_tiling_on_sc=False),
        scratch_types=(
            pltpu.VMEM((ROWS_PER_W, IDX_W), jnp.int32),
            pltpu.VMEM((NBUF, GROWS, D), jnp.float32),
            pltpu.VMEM((UPW,), jnp.int32),
            pltpu.VMEM((UPW, D), jnp.float32),
            pltpu.SemaphoreType.DMA((NBUF,)),
            pltpu.SemaphoreType.DMA((NBUF,)),
            pltpu.SemaphoreType.DMA,
        ),
    )
    def k(item_hbm, idx_hbm, user_hbm, nodes_hbm, g_hbm, u_hbm,
          idx_v, bufs, uidx_v, urows_v, gsems, osems, usem):
        wid = lax.axis_index("s") * NC + lax.axis_index("c")
        pltpu.sync_copy(idx_hbm.at[pl.ds(wid * ROWS_PER_W, ROWS_PER_W)], idx_v)
        pltpu.sync_copy(nodes_hbm.at[wid], uidx_v)
        ucp = pltpu.async_copy(user_hbm.at[uidx_v], urows_v, usem)
        obase = wid * EPW

        def fire_g(g, b):
            for j in range(GROUP):
                pltpu.async_copy(
                    item_hbm.at[idx_v.at[g * GROUP + j]],
                    bufs.at[b, pl.ds(j * IDX_W, IDX_W)], gsems.at[b])

        def drain_g(b):
            pltpu.make_async_copy(
                item_hbm.at[pl.ds(0, GROWS)], bufs.at[b], gsems.at[b]).wait()

        def fire_o(g, b):
            pltpu.async_copy(
                bufs.at[b], g_hbm.at[pl.ds(obase + g * GROWS, GROWS)],
                osems.at[b])

        def drain_o(b):
            pltpu.make_async_copy(
                bufs.at[b], g_hbm.at[pl.ds(obase, GROWS)], osems.at[b]).wait()

        fire_g(0, 0)
        fire_g(1, 1)
        drain_g(0)
        fire_o(0, 0)
        fire_g(2, 2)

        def body(g, carry):
            b = lax.rem(g, NBUF)
            bp = lax.rem(g + 2, NBUF)
            drain_g(b)
            fire_o(g, b)
            drain_o(bp)
            fire_g(g + 2, bp)
            return carry

        lax.fori_loop(1, NG - 2, body, 0)
        drain_g((NG - 2) % NBUF)
        fire_o(NG - 2, (NG - 2) % NBUF)
        drain_o((NG - 3) % NBUF)
        drain_g((NG - 1) % NBUF)
        fire_o(NG - 1, (NG - 1) % NBUF)
        drain_o((NG - 2) % NBUF)
        drain_o((NG - 1) % NBUF)
        ucp.wait()
        pltpu.sync_copy(urows_v, u_hbm.at[pl.ds(wid * UPW, UPW)])

    return k(item_emb, idx2, user_emb, nodes2)


def _tc_body(g_ref, rid_ref, u_ref, w1_ref, w1b_ref, w2_ref, w2b_ref,
             wa1_ref, wa1b_ref, wa2_ref, wa2b_ref, wa3_ref, wa3b_ref,
             remb_ref, out_ref):
    f32 = jnp.float32
    g = g_ref[...]
    ids = rid_ref[...]
    w1a = w1_ref[0:D, :]
    r1 = jnp.dot(remb_ref[...], w1_ref[D:2 * D, :],
                 preferred_element_type=f32)
    oh = jnp.where(lax.broadcasted_iota(jnp.int32, (NBLK, 8), 1) == ids,
                   f32(1.0), f32(0.0))
    rc = jnp.dot(oh, r1, preferred_element_type=f32)

    t = jnp.maximum(jnp.dot(g, w1a, preferred_element_type=f32)
                    + rc + w1b_ref[...], 0.0)
    uv_r = jnp.maximum(jnp.dot(t, w2_ref[...], preferred_element_type=f32)
                       + w2b_ref[...], 0.0)

    self_c = jnp.dot(u_ref[...], wa1_ref[D:2 * D, :],
                     preferred_element_type=f32)
    h1 = jnp.dot(uv_r, wa1_ref[0:D, :], preferred_element_type=f32)
    h = jnp.maximum(h1.reshape(BB, DEGP, D) + self_c[:, None, :]
                    + wa1b_ref[...][None, :, :], 0.0)
    h2 = jnp.maximum(jnp.dot(h.reshape(NBLK, D), wa2_ref[...],
                             preferred_element_type=f32)
                     + wa2b_ref[...], 0.0)
    logits = (jnp.dot(h2, wa3_ref[...], preferred_element_type=f32)
              + wa3b_ref[...])

    l3 = logits.reshape(BB, DEGP, 1)
    pos = lax.broadcasted_iota(jnp.int32, (BB, DEGP, 1), 1)
    l3 = jnp.where(pos < DEG, l3, f32(-1e30))
    m = jnp.max(l3, axis=1, keepdims=True)
    e = jnp.exp(l3 - m)
    s = jnp.sum(e, axis=1)
    num = jnp.sum(uv_r.reshape(BB, DEGP, D) * e, axis=1)
    out_ref[...] = num / s


def _tc_mlp(g, rid, u, w1_w, w1_b, w2_w, w2_b, wa1_w, wa1_b, wa2_w, wa2_b,
            wa3r, wa3_b, remb):
    grid = (B // BB,)
    full = lambda shape: pl.BlockSpec(shape, lambda i: (0, 0))
    return pl.pallas_call(
        _tc_body,
        grid=grid,
        in_specs=[
            pl.BlockSpec((NBLK, D), lambda i: (i, 0)),
            pl.BlockSpec((NBLK, 1), lambda i: (i, 0)),
            pl.BlockSpec((BB, D), lambda i: (i, 0)),
            full((2 * D, D)), full((1, D)),
            full((D, D)), full((1, D)),
            full((2 * D, D)), full((1, D)),
            full((D, D)), full((1, D)),
            full((D, 1)), full((1, 1)),
            full((8, D)),
        ],
        out_specs=pl.BlockSpec((BB, D), lambda i: (i, 0)),
        out_shape=jax.ShapeDtypeStruct((B, D), jnp.float32),
    )(g, rid, u, w1_w, w1_b, w2_w, w2_b, wa1_w, wa1_b, wa2_w, wa2_b,
      wa3r, wa3_b, remb)


def kernel(nodes, uv_adjacency, ratings, user_emb, item_emb, rating_emb,
           w1_w, w1_b, w2_w, w2_b, wa1_w, wa1_b, wa2_w, wa2_b, wa3_w, wa3_b):
    npad = DEGP - DEG
    pad_idx = (lax.broadcasted_iota(jnp.int32, (B, npad), 0) * npad
               + lax.broadcasted_iota(jnp.int32, (B, npad), 1))
    adj_p = jnp.concatenate([uv_adjacency.astype(jnp.int32), pad_idx], axis=1)
    idx2 = adj_p.reshape(B * DEGP // IDX_W, IDX_W)
    rat_p = jnp.pad(ratings.astype(jnp.int32), ((0, 0), (0, DEGP - DEG)))
    rid = rat_p.reshape(B * DEGP, 1)
    nodes2 = nodes.astype(jnp.int32).reshape(NW, UPW)

    g, u = _sc_gather(item_emb, idx2, user_emb, nodes2)

    remb = jnp.pad(rating_emb, ((0, 3), (0, 0)))
    return _tc_mlp(
        g, rid, u,
        w1_w, w1_b.reshape(1, D),
        w2_w, w2_b.reshape(1, D),
        wa1_w, wa1_b.reshape(1, D),
        wa2_w, wa2_b.reshape(1, D),
        wa3_w, wa3_b.reshape(1, 1),
        remb)
